# baseline (speedup 1.0000x reference)
import functools

import jax
import jax.numpy as jnp
from jax import lax
from jax.experimental import pallas as pl
from jax.experimental.pallas import tpu as pltpu
from jax.experimental.pallas import tpu_sc as plsc

K_SC = 1
NSUB = 16
LANES = 16
ROWS = 2


def _gelu16(v):
    z = v * 0.7071067811865476
    az = jnp.abs(z)
    t = 1.0 / (1.0 + 0.3275911 * az)
    poly = ((((1.061405429 * t - 1.453152027) * t + 1.421413741) * t
             - 0.284496736) * t + 0.254829592) * t
    e = poly * jnp.exp(-az * az)
    erf_abs = 1.0 - e
    erf = jnp.where(z >= 0.0, erf_abs, -erf_abs)
    return 0.5 * v * (1.0 + erf)


def _sc_expert_kernel(xt_hbm, w1_hbm, w2_hbm, ytp_hbm, ht_hbm,
                      xt_v, w1_v, ht_v, hf_v, w2_v, yt_v):
    c = lax.axis_index("c")
    s = lax.axis_index("s")

    IH, H = w1_hbm.shape[1], w1_hbm.shape[2]
    CAP = 2 * LANES
    RI = IH // NSUB
    RH = H // NSUB
    FC = hf_v.shape[0] // CAP

    zero = jnp.zeros((LANES,), jnp.float32)

    def pair_accumulate(w_v, rbase, act_v, n16, init4):
        def body(k, accs):
            a00, a01, a10, a11 = accs
            f0 = k * LANES
            wv0 = w_v[rbase, pl.ds(f0, LANES)]
            wv1 = w_v[rbase + 1, pl.ds(f0, LANES)]
            for j in range(LANES):
                x0 = act_v[pl.ds((f0 + j) * CAP, LANES)]
                x1 = act_v[pl.ds((f0 + j) * CAP + LANES, LANES)]
                w0 = wv0[j]
                w1 = wv1[j]
                a00 = a00 + w0 * x0
                a01 = a01 + w0 * x1
                a10 = a10 + w1 * x0
                a11 = a11 + w1 * x1
            return (a00, a01, a10, a11)

        return lax.fori_loop(0, n16, body, init4)

    pltpu.sync_copy(xt_hbm.at[c], xt_v)

    W1C = w1_v.shape[0]
    for cc in range(RI // W1C):
        pltpu.sync_copy(
            w1_hbm.at[c, pl.ds(s * RI + cc * W1C, W1C), :], w1_v)

        def row_body(r2, _, cc=cc):
            rbase = r2 * 2
            h00, h01, h10, h11 = pair_accumulate(
                w1_v, rbase, xt_v, H // LANES, (zero, zero, zero, zero))
            base = (cc * W1C + rbase) * CAP
            ht_v[pl.ds(base, LANES)] = _gelu16(h00)
            ht_v[pl.ds(base + LANES, LANES)] = _gelu16(h01)
            ht_v[pl.ds(base + CAP, LANES)] = _gelu16(h10)
            ht_v[pl.ds(base + CAP + LANES, LANES)] = _gelu16(h11)
            return 0

        lax.fori_loop(0, W1C // 2, row_body, 0)

    pltpu.sync_copy(ht_v, ht_hbm.at[c, pl.ds(s * RI * CAP, RI * CAP)])
    plsc.subcore_barrier()

    def zero_body(r, _):
        yt_v[pl.ds(r * CAP, LANES)] = zero
        yt_v[pl.ds(r * CAP + LANES, LANES)] = zero
        return 0

    lax.fori_loop(0, RH, zero_body, 0)

    for fc in range(IH // FC):
        pltpu.sync_copy(
            ht_hbm.at[c, pl.ds(fc * FC * CAP, FC * CAP)], hf_v)
        pltpu.sync_copy(
            w2_hbm.at[0, pl.ds(s * RH, RH), c, pl.ds(fc * FC, FC)],
            w2_v)

        def row2_body(r2, _):
            rbase = r2 * 2
            i0 = yt_v[pl.ds(rbase * CAP, LANES)]
            i1 = yt_v[pl.ds(rbase * CAP + LANES, LANES)]
            i2 = yt_v[pl.ds(rbase * CAP + CAP, LANES)]
            i3 = yt_v[pl.ds(rbase * CAP + CAP + LANES, LANES)]
            y00, y01, y10, y11 = pair_accumulate(
                w2_v, rbase, hf_v, FC // LANES, (i0, i1, i2, i3))
            yt_v[pl.ds(rbase * CAP, LANES)] = y00
            yt_v[pl.ds(rbase * CAP + LANES, LANES)] = y01
            yt_v[pl.ds(rbase * CAP + CAP, LANES)] = y10
            yt_v[pl.ds(rbase * CAP + CAP + LANES, LANES)] = y11
            return 0

        lax.fori_loop(0, RH // 2, row2_body, 0)

    pltpu.sync_copy(yt_v, ytp_hbm.at[c, pl.ds(s * RH * CAP, RH * CAP)])


def _tc_ffn_kernel(x_ref, w1a_ref, w1b_ref, b1_ref, w2a_ref, w2b_ref, b2_ref,
                   o_ref):
    I2 = w1a_ref.shape[2]
    H2 = w2a_ref.shape[2]
    x = x_ref[0]

    def dot_t(a, b):
        return jax.lax.dot_general(
            a, b, (((1,), (1,)), ((), ())), preferred_element_type=jnp.float32
        )

    b1 = b1_ref[0]
    h_a = dot_t(x, w1a_ref[0, 0]) + b1[:, :I2]
    h_b = dot_t(x, w1b_ref[0, 0]) + b1[:, I2:]
    h_a = 0.5 * h_a * (1.0 + jax.lax.erf(h_a * 0.7071067811865476))
    h_b = 0.5 * h_b * (1.0 + jax.lax.erf(h_b * 0.7071067811865476))

    w2a = w2a_ref[0, 0]
    w2b = w2b_ref[0, 0]
    b2 = b2_ref[0]
    y_a = dot_t(h_a, w2a[:, :I2]) + dot_t(h_b, w2a[:, I2:]) + b2[:, :H2]
    y_b = dot_t(h_a, w2b[:, :I2]) + dot_t(h_b, w2b[:, I2:]) + b2[:, H2:]
    o_ref[0] = jnp.concatenate([y_a, y_b], axis=1)


def kernel(inter_state, W1, b1, W2, b2, loads):
    E, I, H = W1.shape
    CAP = inter_state.shape[0] // E
    K = K_SC
    ETC = E - K

    x = inter_state.reshape(E, CAP, H)
    W1s = W1.reshape(E, 2, I // 2, H)
    W2s = W2.reshape(E, 2, H // 2, I)
    b1r = b1.reshape(E, 1, I)
    b2r = b2.reshape(E, 1, H)

    xt = jnp.transpose(x[:K], (0, 2, 1)).reshape(K, H * CAP)
    mesh = plsc.VectorSubcoreMesh(core_axis_name="c", subcore_axis_name="s")
    FC = 256
    sc_call = functools.partial(
        pl.kernel,
        mesh=mesh,
        out_type=[
            jax.ShapeDtypeStruct((2, H * CAP), jnp.float32),
            jax.ShapeDtypeStruct((2, I // 2 * CAP), jnp.float32),
        ],
        scratch_types=[
            pltpu.VMEM((H * CAP,), jnp.float32),
            pltpu.VMEM((32, H), jnp.float32),
            pltpu.VMEM((I // 2 // NSUB * CAP,), jnp.float32),
            pltpu.VMEM((FC * CAP,), jnp.float32),
            pltpu.VMEM((H // NSUB, FC), jnp.float32),
            pltpu.VMEM((H // NSUB * CAP,), jnp.float32),
        ],
    )(_sc_expert_kernel)
    ytp, _ = sc_call(jnp.concatenate([xt, xt], axis=0),
                     W1.reshape(E * 2, I // 2, H),
                     W2.reshape(E, H, 2, I // 2))
    yt_sc = (ytp[0] + ytp[1]).reshape(H, CAP)
    y_sc = jnp.transpose(yt_sc, (1, 0)).reshape(K, CAP, H)

    y_tc = pl.pallas_call(
        _tc_ffn_kernel,
        grid=(ETC,),
        in_specs=[
            pl.BlockSpec((1, CAP, H), lambda e: (e + K, 0, 0)),
            pl.BlockSpec((1, 1, I // 2, H), lambda e: (e + K, 0, 0, 0)),
            pl.BlockSpec((1, 1, I // 2, H), lambda e: (e + K, 1, 0, 0)),
            pl.BlockSpec((1, 1, I), lambda e: (e + K, 0, 0)),
            pl.BlockSpec((1, 1, H // 2, I), lambda e: (e + K, 0, 0, 0)),
            pl.BlockSpec((1, 1, H // 2, I), lambda e: (e + K, 1, 0, 0)),
            pl.BlockSpec((1, 1, H), lambda e: (e + K, 0, 0)),
        ],
        out_specs=pl.BlockSpec((1, CAP, H), lambda e: (e, 0, 0)),
        out_shape=jax.ShapeDtypeStruct((ETC, CAP, H), jnp.float32),
        compiler_params=pltpu.CompilerParams(
            dimension_semantics=("parallel",),
        ),
    )(x, W1s, W1s, b1r, W2s, W2s, b2r)

    out = jnp.concatenate([y_sc, y_tc], axis=0)
    return out.reshape(E * CAP, H)


# speedup vs baseline: 5.1456x; 5.1456x over previous
import functools

import jax
import jax.numpy as jnp
from jax import lax
from jax.experimental import pallas as pl
from jax.experimental.pallas import tpu as pltpu
from jax.experimental.pallas import tpu_sc as plsc

K_SC = 1
NSUB = 16
LANES = 16
ROWS = 2


def _gelu16(v):
    z = v * 0.7071067811865476
    az = jnp.abs(z)
    t = 1.0 / (1.0 + 0.3275911 * az)
    poly = ((((1.061405429 * t - 1.453152027) * t + 1.421413741) * t
             - 0.284496736) * t + 0.254829592) * t
    e = poly * jnp.exp(-az * az)
    erf_abs = 1.0 - e
    erf = jnp.where(z >= 0.0, erf_abs, -erf_abs)
    return 0.5 * v * (1.0 + erf)


def _sc_expert_kernel(xt_hbm, w1_hbm, w2_hbm, ytp_hbm, ht_hbm,
                      xt_v, w1_v, ht_v, hf_v, w2_v, yt_v):
    c = lax.axis_index("c")
    s = lax.axis_index("s")

    IH, H = w1_hbm.shape[1], w1_hbm.shape[2]
    CAP = 2 * LANES
    RI = IH // NSUB
    RH = H // NSUB
    FC = hf_v.shape[0] // CAP

    zero = jnp.zeros((LANES,), jnp.float32)

    def pair_accumulate(w_v, rbase, act_v, n16, init4):
        def body(k, accs):
            a00, a01, a10, a11 = accs
            f0 = k * LANES
            wv0 = w_v[rbase, pl.ds(f0, LANES)]
            wv1 = w_v[rbase + 1, pl.ds(f0, LANES)]
            for j in range(LANES):
                x0 = act_v[pl.ds((f0 + j) * CAP, LANES)]
                x1 = act_v[pl.ds((f0 + j) * CAP + LANES, LANES)]
                w0 = wv0[j]
                w1 = wv1[j]
                a00 = a00 + w0 * x0
                a01 = a01 + w0 * x1
                a10 = a10 + w1 * x0
                a11 = a11 + w1 * x1
            return (a00, a01, a10, a11)

        return lax.fori_loop(0, n16, body, init4)

    pltpu.sync_copy(xt_hbm.at[c], xt_v)

    W1C = w1_v.shape[0]
    for cc in range(RI // W1C):
        pltpu.sync_copy(
            w1_hbm.at[c, pl.ds(s * RI + cc * W1C, W1C), :], w1_v)

        def row_body(r2, _, cc=cc):
            rbase = r2 * 2
            h00, h01, h10, h11 = pair_accumulate(
                w1_v, rbase, xt_v, H // LANES, (zero, zero, zero, zero))
            base = (cc * W1C + rbase) * CAP
            ht_v[pl.ds(base, LANES)] = _gelu16(h00)
            ht_v[pl.ds(base + LANES, LANES)] = _gelu16(h01)
            ht_v[pl.ds(base + CAP, LANES)] = _gelu16(h10)
            ht_v[pl.ds(base + CAP + LANES, LANES)] = _gelu16(h11)
            return 0

        lax.fori_loop(0, W1C // 2, row_body, 0)

    pltpu.sync_copy(ht_v, ht_hbm.at[c, pl.ds(s * RI * CAP, RI * CAP)])
    plsc.subcore_barrier()

    def zero_body(r, _):
        yt_v[pl.ds(r * CAP, LANES)] = zero
        yt_v[pl.ds(r * CAP + LANES, LANES)] = zero
        return 0

    lax.fori_loop(0, RH, zero_body, 0)

    for fc in range(IH // FC):
        pltpu.sync_copy(
            ht_hbm.at[c, pl.ds(fc * FC * CAP, FC * CAP)], hf_v)
        pltpu.sync_copy(
            w2_hbm.at[c, pl.ds(s * RH, RH), pl.ds(fc * FC, FC)],
            w2_v)

        def row2_body(r2, _):
            rbase = r2 * 2
            i0 = yt_v[pl.ds(rbase * CAP, LANES)]
            i1 = yt_v[pl.ds(rbase * CAP + LANES, LANES)]
            i2 = yt_v[pl.ds(rbase * CAP + CAP, LANES)]
            i3 = yt_v[pl.ds(rbase * CAP + CAP + LANES, LANES)]
            y00, y01, y10, y11 = pair_accumulate(
                w2_v, rbase, hf_v, FC // LANES, (i0, i1, i2, i3))
            yt_v[pl.ds(rbase * CAP, LANES)] = y00
            yt_v[pl.ds(rbase * CAP + LANES, LANES)] = y01
            yt_v[pl.ds(rbase * CAP + CAP, LANES)] = y10
            yt_v[pl.ds(rbase * CAP + CAP + LANES, LANES)] = y11
            return 0

        lax.fori_loop(0, RH // 2, row2_body, 0)

    pltpu.sync_copy(yt_v, ytp_hbm.at[c, pl.ds(s * RH * CAP, RH * CAP)])


def _tc_ffn_kernel(x_ref, w1a_ref, w1b_ref, b1_ref, w2a_ref, w2b_ref, b2_ref,
                   o_ref):
    I2 = w1a_ref.shape[2]
    H2 = w2a_ref.shape[2]
    x = x_ref[0]

    def dot_t(a, b):
        return jax.lax.dot_general(
            a, b, (((1,), (1,)), ((), ())), preferred_element_type=jnp.float32
        )

    b1 = b1_ref[0]
    h_a = dot_t(x, w1a_ref[0, 0]) + b1[:, :I2]
    h_b = dot_t(x, w1b_ref[0, 0]) + b1[:, I2:]
    h_a = 0.5 * h_a * (1.0 + jax.lax.erf(h_a * 0.7071067811865476))
    h_b = 0.5 * h_b * (1.0 + jax.lax.erf(h_b * 0.7071067811865476))

    w2a = w2a_ref[0, 0]
    w2b = w2b_ref[0, 0]
    b2 = b2_ref[0]
    y_a = dot_t(h_a, w2a[:, :I2]) + dot_t(h_b, w2a[:, I2:]) + b2[:, :H2]
    y_b = dot_t(h_a, w2b[:, :I2]) + dot_t(h_b, w2b[:, I2:]) + b2[:, H2:]
    o_ref[0] = jnp.concatenate([y_a, y_b], axis=1)


def kernel(inter_state, W1, b1, W2, b2, loads):
    E, I, H = W1.shape
    CAP = inter_state.shape[0] // E
    K = K_SC
    ETC = E - K

    x = inter_state.reshape(E, CAP, H)
    W1s = W1.reshape(E, 2, I // 2, H)
    W2s = W2.reshape(E, 2, H // 2, I)
    b1r = b1.reshape(E, 1, I)
    b2r = b2.reshape(E, 1, H)

    xt = jnp.transpose(x[:K], (0, 2, 1)).reshape(K, H * CAP)
    mesh = plsc.VectorSubcoreMesh(core_axis_name="c", subcore_axis_name="s")
    FC = 256
    sc_call = functools.partial(
        pl.kernel,
        mesh=mesh,
        out_type=[
            jax.ShapeDtypeStruct((2, H * CAP), jnp.float32),
            jax.ShapeDtypeStruct((2, I // 2 * CAP), jnp.float32),
        ],
        scratch_types=[
            pltpu.VMEM((H * CAP,), jnp.float32),
            pltpu.VMEM((32, H), jnp.float32),
            pltpu.VMEM((I // 2 // NSUB * CAP,), jnp.float32),
            pltpu.VMEM((FC * CAP,), jnp.float32),
            pltpu.VMEM((H // NSUB, FC), jnp.float32),
            pltpu.VMEM((H // NSUB * CAP,), jnp.float32),
        ],
    )(_sc_expert_kernel)
    W2sc = W2[0].reshape(H, 2, I // 2).transpose(1, 0, 2)
    ytp, _ = sc_call(jnp.concatenate([xt, xt], axis=0),
                     W1.reshape(E * 2, I // 2, H),
                     W2sc)
    yt_sc = (ytp[0] + ytp[1]).reshape(H, CAP)
    y_sc = jnp.transpose(yt_sc, (1, 0)).reshape(K, CAP, H)

    y_tc = pl.pallas_call(
        _tc_ffn_kernel,
        grid=(ETC,),
        in_specs=[
            pl.BlockSpec((1, CAP, H), lambda e: (e + K, 0, 0)),
            pl.BlockSpec((1, 1, I // 2, H), lambda e: (e + K, 0, 0, 0)),
            pl.BlockSpec((1, 1, I // 2, H), lambda e: (e + K, 1, 0, 0)),
            pl.BlockSpec((1, 1, I), lambda e: (e + K, 0, 0)),
            pl.BlockSpec((1, 1, H // 2, I), lambda e: (e + K, 0, 0, 0)),
            pl.BlockSpec((1, 1, H // 2, I), lambda e: (e + K, 1, 0, 0)),
            pl.BlockSpec((1, 1, H), lambda e: (e + K, 0, 0)),
        ],
        out_specs=pl.BlockSpec((1, CAP, H), lambda e: (e, 0, 0)),
        out_shape=jax.ShapeDtypeStruct((ETC, CAP, H), jnp.float32),
        compiler_params=pltpu.CompilerParams(
            dimension_semantics=("parallel",),
        ),
    )(x, W1s, W1s, b1r, W2s, W2s, b2r)

    out = jnp.concatenate([y_sc, y_tc], axis=0)
    return out.reshape(E * CAP, H)


# speedup vs baseline: 5.1931x; 1.0092x over previous
import functools

import jax
import jax.numpy as jnp
from jax import lax
from jax.experimental import pallas as pl
from jax.experimental.pallas import tpu as pltpu
from jax.experimental.pallas import tpu_sc as plsc

K_SC = 1
NSUB = 16
LANES = 16
ROWS = 2


def _gelu16(v):
    z = v * 0.7071067811865476
    az = jnp.abs(z)
    t = 1.0 / (1.0 + 0.3275911 * az)
    poly = ((((1.061405429 * t - 1.453152027) * t + 1.421413741) * t
             - 0.284496736) * t + 0.254829592) * t
    e = poly * jnp.exp(-az * az)
    erf_abs = 1.0 - e
    erf = jnp.where(z >= 0.0, erf_abs, -erf_abs)
    return 0.5 * v * (1.0 + erf)


def _sc_expert_kernel(xt_hbm, w1_hbm, w2_hbm, ytp_hbm,
                      xt_v, w1_v, ht_v, hf_v, w2_v, yt_v, hsh_v):
    c = lax.axis_index("c")
    s = lax.axis_index("s")

    IH, H = w1_hbm.shape[1], w1_hbm.shape[2]
    CAP = 2 * LANES
    RI = IH // NSUB
    RH = H // NSUB
    FC = hf_v.shape[0] // CAP

    zero = jnp.zeros((LANES,), jnp.float32)

    def pair_accumulate(w_v, rbase, act_v, n16, init4):
        def body(k, accs):
            a00, a01, a10, a11 = accs
            f0 = k * LANES
            wv0 = w_v[rbase, pl.ds(f0, LANES)]
            wv1 = w_v[rbase + 1, pl.ds(f0, LANES)]
            for j in range(LANES):
                x0 = act_v[pl.ds((f0 + j) * CAP, LANES)]
                x1 = act_v[pl.ds((f0 + j) * CAP + LANES, LANES)]
                w0 = wv0[j]
                w1 = wv1[j]
                a00 = a00 + w0 * x0
                a01 = a01 + w0 * x1
                a10 = a10 + w1 * x0
                a11 = a11 + w1 * x1
            return (a00, a01, a10, a11)

        return lax.fori_loop(0, n16, body, init4)

    pltpu.sync_copy(xt_hbm.at[c], xt_v)

    W1C = w1_v.shape[0]
    for cc in range(RI // W1C):
        pltpu.sync_copy(
            w1_hbm.at[c, pl.ds(s * RI + cc * W1C, W1C), :], w1_v)

        def row_body(r2, _, cc=cc):
            rbase = r2 * 2
            h00, h01, h10, h11 = pair_accumulate(
                w1_v, rbase, xt_v, H // LANES, (zero, zero, zero, zero))
            base = (cc * W1C + rbase) * CAP
            ht_v[pl.ds(base, LANES)] = _gelu16(h00)
            ht_v[pl.ds(base + LANES, LANES)] = _gelu16(h01)
            ht_v[pl.ds(base + CAP, LANES)] = _gelu16(h10)
            ht_v[pl.ds(base + CAP + LANES, LANES)] = _gelu16(h11)
            return 0

        lax.fori_loop(0, W1C // 2, row_body, 0)

    pltpu.sync_copy(ht_v, hsh_v.at[pl.ds(s * RI * CAP, RI * CAP)])
    plsc.subcore_barrier()

    def zero_body(r, _):
        yt_v[pl.ds(r * CAP, LANES)] = zero
        yt_v[pl.ds(r * CAP + LANES, LANES)] = zero
        return 0

    lax.fori_loop(0, RH, zero_body, 0)

    for fc in range(IH // FC):
        pltpu.sync_copy(hsh_v.at[pl.ds(fc * FC * CAP, FC * CAP)], hf_v)
        pltpu.sync_copy(
            w2_hbm.at[c, pl.ds(s * RH, RH), pl.ds(fc * FC, FC)],
            w2_v)

        def row2_body(r2, _):
            rbase = r2 * 2
            i0 = yt_v[pl.ds(rbase * CAP, LANES)]
            i1 = yt_v[pl.ds(rbase * CAP + LANES, LANES)]
            i2 = yt_v[pl.ds(rbase * CAP + CAP, LANES)]
            i3 = yt_v[pl.ds(rbase * CAP + CAP + LANES, LANES)]
            y00, y01, y10, y11 = pair_accumulate(
                w2_v, rbase, hf_v, FC // LANES, (i0, i1, i2, i3))
            yt_v[pl.ds(rbase * CAP, LANES)] = y00
            yt_v[pl.ds(rbase * CAP + LANES, LANES)] = y01
            yt_v[pl.ds(rbase * CAP + CAP, LANES)] = y10
            yt_v[pl.ds(rbase * CAP + CAP + LANES, LANES)] = y11
            return 0

        lax.fori_loop(0, RH // 2, row2_body, 0)

    pltpu.sync_copy(yt_v, ytp_hbm.at[c, pl.ds(s * RH * CAP, RH * CAP)])


def _tc_ffn_kernel(x_ref, w1a_ref, w1b_ref, b1_ref, w2a_ref, w2b_ref, b2_ref,
                   o_ref):
    I2 = w1a_ref.shape[2]
    H2 = w2a_ref.shape[2]
    x = x_ref[0]

    def dot_t(a, b):
        return jax.lax.dot_general(
            a, b, (((1,), (1,)), ((), ())), preferred_element_type=jnp.float32
        )

    b1 = b1_ref[0]
    h_a = dot_t(x, w1a_ref[0, 0]) + b1[:, :I2]
    h_b = dot_t(x, w1b_ref[0, 0]) + b1[:, I2:]
    h_a = 0.5 * h_a * (1.0 + jax.lax.erf(h_a * 0.7071067811865476))
    h_b = 0.5 * h_b * (1.0 + jax.lax.erf(h_b * 0.7071067811865476))

    w2a = w2a_ref[0, 0]
    w2b = w2b_ref[0, 0]
    b2 = b2_ref[0]
    y_a = dot_t(h_a, w2a[:, :I2]) + dot_t(h_b, w2a[:, I2:]) + b2[:, :H2]
    y_b = dot_t(h_a, w2b[:, :I2]) + dot_t(h_b, w2b[:, I2:]) + b2[:, H2:]
    o_ref[0] = jnp.concatenate([y_a, y_b], axis=1)


def kernel(inter_state, W1, b1, W2, b2, loads):
    E, I, H = W1.shape
    CAP = inter_state.shape[0] // E
    K = K_SC
    ETC = E - K

    x = inter_state.reshape(E, CAP, H)
    W1s = W1.reshape(E, 2, I // 2, H)
    W2s = W2.reshape(E, 2, H // 2, I)
    b1r = b1.reshape(E, 1, I)
    b2r = b2.reshape(E, 1, H)

    xt = jnp.transpose(x[:K], (0, 2, 1)).reshape(K, H * CAP)
    mesh = plsc.VectorSubcoreMesh(core_axis_name="c", subcore_axis_name="s")
    FC = 256
    sc_call = functools.partial(
        pl.kernel,
        mesh=mesh,
        out_type=[
            jax.ShapeDtypeStruct((2, H * CAP), jnp.float32),
        ],
        scratch_types=[
            pltpu.VMEM((H * CAP,), jnp.float32),
            pltpu.VMEM((32, H), jnp.float32),
            pltpu.VMEM((I // 2 // NSUB * CAP,), jnp.float32),
            pltpu.VMEM((FC * CAP,), jnp.float32),
            pltpu.VMEM((H // NSUB, FC), jnp.float32),
            pltpu.VMEM((H // NSUB * CAP,), jnp.float32),
            pltpu.VMEM_SHARED((I // 2 * CAP,), jnp.float32),
        ],
    )(_sc_expert_kernel)
    W2sc = W2[0].reshape(H, 2, I // 2).transpose(1, 0, 2)
    (ytp,) = sc_call(jnp.concatenate([xt, xt], axis=0),
                     W1.reshape(E * 2, I // 2, H),
                     W2sc)
    yt_sc = (ytp[0] + ytp[1]).reshape(H, CAP)
    y_sc = jnp.transpose(yt_sc, (1, 0)).reshape(K, CAP, H)

    y_tc = pl.pallas_call(
        _tc_ffn_kernel,
        grid=(ETC,),
        in_specs=[
            pl.BlockSpec((1, CAP, H), lambda e: (e + K, 0, 0)),
            pl.BlockSpec((1, 1, I // 2, H), lambda e: (e + K, 0, 0, 0)),
            pl.BlockSpec((1, 1, I // 2, H), lambda e: (e + K, 1, 0, 0)),
            pl.BlockSpec((1, 1, I), lambda e: (e + K, 0, 0)),
            pl.BlockSpec((1, 1, H // 2, I), lambda e: (e + K, 0, 0, 0)),
            pl.BlockSpec((1, 1, H // 2, I), lambda e: (e + K, 1, 0, 0)),
            pl.BlockSpec((1, 1, H), lambda e: (e + K, 0, 0)),
        ],
        out_specs=pl.BlockSpec((1, CAP, H), lambda e: (e, 0, 0)),
        out_shape=jax.ShapeDtypeStruct((ETC, CAP, H), jnp.float32),
        compiler_params=pltpu.CompilerParams(
            dimension_semantics=("parallel",),
        ),
    )(x, W1s, W1s, b1r, W2s, W2s, b2r)

    out = jnp.concatenate([y_sc, y_tc], axis=0)
    return out.reshape(E * CAP, H)


# speedup vs baseline: 5.2510x; 1.0112x over previous
import functools

import jax
import jax.numpy as jnp
from jax import lax
from jax.experimental import pallas as pl
from jax.experimental.pallas import tpu as pltpu
from jax.experimental.pallas import tpu_sc as plsc

K_SC = 1
NSUB = 16
LANES = 16
ROWS = 2


def _gelu16(v):
    z = v * 0.7071067811865476
    az = jnp.abs(z)
    t = 1.0 / (1.0 + 0.3275911 * az)
    poly = ((((1.061405429 * t - 1.453152027) * t + 1.421413741) * t
             - 0.284496736) * t + 0.254829592) * t
    e = poly * jnp.exp(-az * az)
    erf_abs = 1.0 - e
    erf = jnp.where(z >= 0.0, erf_abs, -erf_abs)
    return 0.5 * v * (1.0 + erf)


def _sc_expert_kernel(xt_hbm, w1_hbm, w2_hbm, ytp_hbm,
                      xt_v, w1_v, ht_v, hf_v, w2_v, yt_v, hsh_v):
    c = lax.axis_index("c")
    s = lax.axis_index("s")

    IH, H = w1_hbm.shape[1], w1_hbm.shape[2]
    CAP = 2 * LANES
    RI = IH // NSUB
    RH = H // NSUB
    FC = hf_v.shape[0] // CAP

    zero = jnp.zeros((LANES,), jnp.float32)

    def pair_accumulate(w_v, rbase, act_v, n16, init4):
        def body(k, accs):
            a00, a01, a10, a11 = accs
            f0 = k * LANES
            wv0 = w_v[rbase, pl.ds(f0, LANES)]
            wv1 = w_v[rbase + 1, pl.ds(f0, LANES)]
            for j in range(LANES):
                x0 = act_v[pl.ds((f0 + j) * CAP, LANES)]
                x1 = act_v[pl.ds((f0 + j) * CAP + LANES, LANES)]
                w0 = wv0[j]
                w1 = wv1[j]
                a00 = a00 + w0 * x0
                a01 = a01 + w0 * x1
                a10 = a10 + w1 * x0
                a11 = a11 + w1 * x1
            return (a00, a01, a10, a11)

        return lax.fori_loop(0, n16, body, init4)

    pltpu.sync_copy(xt_hbm.at[c], xt_v)

    W1C = w1_v.shape[0]
    for cc in range(RI // W1C):
        pltpu.sync_copy(
            w1_hbm.at[c, pl.ds(s * RI + cc * W1C, W1C), :], w1_v)

        def row_body(r2, _, cc=cc):
            rbase = r2 * 2
            h00, h01, h10, h11 = pair_accumulate(
                w1_v, rbase, xt_v, H // LANES, (zero, zero, zero, zero))
            base = (cc * W1C + rbase) * CAP
            ht_v[pl.ds(base, LANES)] = _gelu16(h00)
            ht_v[pl.ds(base + LANES, LANES)] = _gelu16(h01)
            ht_v[pl.ds(base + CAP, LANES)] = _gelu16(h10)
            ht_v[pl.ds(base + CAP + LANES, LANES)] = _gelu16(h11)
            return 0

        lax.fori_loop(0, W1C // 2, row_body, 0)

    pltpu.sync_copy(ht_v, hsh_v.at[pl.ds(s * RI * CAP, RI * CAP)])
    plsc.subcore_barrier()

    def zero_body(r, _):
        yt_v[pl.ds(r * CAP, LANES)] = zero
        yt_v[pl.ds(r * CAP + LANES, LANES)] = zero
        return 0

    lax.fori_loop(0, RH, zero_body, 0)

    for fc in range(IH // FC):
        pltpu.sync_copy(hsh_v.at[pl.ds(fc * FC * CAP, FC * CAP)], hf_v)
        pltpu.sync_copy(
            w2_hbm.at[c, pl.ds(s * RH, RH), pl.ds(fc * FC, FC)],
            w2_v)

        def row2_body(r2, _):
            rbase = r2 * 2
            i0 = yt_v[pl.ds(rbase * CAP, LANES)]
            i1 = yt_v[pl.ds(rbase * CAP + LANES, LANES)]
            i2 = yt_v[pl.ds(rbase * CAP + CAP, LANES)]
            i3 = yt_v[pl.ds(rbase * CAP + CAP + LANES, LANES)]
            y00, y01, y10, y11 = pair_accumulate(
                w2_v, rbase, hf_v, FC // LANES, (i0, i1, i2, i3))
            yt_v[pl.ds(rbase * CAP, LANES)] = y00
            yt_v[pl.ds(rbase * CAP + LANES, LANES)] = y01
            yt_v[pl.ds(rbase * CAP + CAP, LANES)] = y10
            yt_v[pl.ds(rbase * CAP + CAP + LANES, LANES)] = y11
            return 0

        lax.fori_loop(0, RH // 2, row2_body, 0)

    pltpu.sync_copy(yt_v, ytp_hbm.at[c, pl.ds(s * RH * CAP, RH * CAP)])


def _tc_ffn_kernel(x_ref, w1a_ref, w1b_ref, b1_ref, w2a_ref, w2b_ref, b2_ref,
                   o_ref):
    I2 = w1a_ref.shape[2]
    H2 = w2a_ref.shape[2]
    x = x_ref[0]

    def dot_t(a, b):
        return jax.lax.dot_general(
            a, b, (((1,), (1,)), ((), ())), preferred_element_type=jnp.float32
        )

    b1 = b1_ref[0]
    h_a = dot_t(x, w1a_ref[0, 0]) + b1[:, :I2]
    h_b = dot_t(x, w1b_ref[0, 0]) + b1[:, I2:]
    h_a = 0.5 * h_a * (1.0 + jax.lax.erf(h_a * 0.7071067811865476))
    h_b = 0.5 * h_b * (1.0 + jax.lax.erf(h_b * 0.7071067811865476))

    w2a = w2a_ref[0, 0]
    w2b = w2b_ref[0, 0]
    b2 = b2_ref[0]
    y_a = dot_t(h_a, w2a[:, :I2]) + dot_t(h_b, w2a[:, I2:]) + b2[:, :H2]
    y_b = dot_t(h_a, w2b[:, :I2]) + dot_t(h_b, w2b[:, I2:]) + b2[:, H2:]
    o_ref[0] = jnp.concatenate([y_a, y_b], axis=1)


def kernel(inter_state, W1, b1, W2, b2, loads):
    E, I, H = W1.shape
    CAP = inter_state.shape[0] // E
    K = K_SC
    ETC = E - K

    x = inter_state.reshape(E, CAP, H)
    W1s = W1.reshape(E, 2, I // 2, H)
    W2s = W2.reshape(E, 2, H // 2, I)
    b1r = b1.reshape(E, 1, I)
    b2r = b2.reshape(E, 1, H)

    xt = jnp.transpose(x[:K], (0, 2, 1)).reshape(K, H * CAP)
    mesh = plsc.VectorSubcoreMesh(core_axis_name="c", subcore_axis_name="s",
                                  num_cores=2)
    FC = 256
    sc_call = functools.partial(
        pl.kernel,
        mesh=mesh,
        out_type=[
            jax.ShapeDtypeStruct((2, H * CAP), jnp.float32),
        ],
        scratch_types=[
            pltpu.VMEM((H * CAP,), jnp.float32),
            pltpu.VMEM((32, H), jnp.float32),
            pltpu.VMEM((I // 2 // NSUB * CAP,), jnp.float32),
            pltpu.VMEM((FC * CAP,), jnp.float32),
            pltpu.VMEM((H // NSUB, FC), jnp.float32),
            pltpu.VMEM((H // NSUB * CAP,), jnp.float32),
            pltpu.VMEM_SHARED((I // 2 * CAP,), jnp.float32),
        ],
    )(_sc_expert_kernel)
    W2sc = W2[0].reshape(H, 2, I // 2).transpose(1, 0, 2)
    (ytp,) = sc_call(jnp.concatenate([xt, xt], axis=0),
                     W1.reshape(E * 2, I // 2, H),
                     W2sc)
    yt_sc = (ytp[0] + ytp[1]).reshape(H, CAP)
    y_sc = jnp.transpose(yt_sc, (1, 0)).reshape(K, CAP, H)

    y_tc = pl.pallas_call(
        _tc_ffn_kernel,
        grid=(ETC,),
        in_specs=[
            pl.BlockSpec((1, CAP, H), lambda e: (e + K, 0, 0)),
            pl.BlockSpec((1, 1, I // 2, H), lambda e: (e + K, 0, 0, 0)),
            pl.BlockSpec((1, 1, I // 2, H), lambda e: (e + K, 1, 0, 0)),
            pl.BlockSpec((1, 1, I), lambda e: (e + K, 0, 0)),
            pl.BlockSpec((1, 1, H // 2, I), lambda e: (e + K, 0, 0, 0)),
            pl.BlockSpec((1, 1, H // 2, I), lambda e: (e + K, 1, 0, 0)),
            pl.BlockSpec((1, 1, H), lambda e: (e + K, 0, 0)),
        ],
        out_specs=pl.BlockSpec((1, CAP, H), lambda e: (e, 0, 0)),
        out_shape=jax.ShapeDtypeStruct((ETC, CAP, H), jnp.float32),
        compiler_params=pltpu.CompilerParams(
            dimension_semantics=("parallel",),
        ),
    )(x, W1s, W1s, b1r, W2s, W2s, b2r)

    out = jnp.concatenate([y_sc, y_tc], axis=0)
    return out.reshape(E * CAP, H)


# speedup vs baseline: 5.9964x; 1.1419x over previous
import jax
import jax.numpy as jnp
from jax.experimental import pallas as pl
from jax.experimental.pallas import tpu as pltpu


def _ffn_kernel(x_ref, w1a_ref, w1b_ref, b1_ref, w2a_ref, w2b_ref, b2_ref,
                o_ref):
    CAP = x_ref.shape[1]
    I2 = w1a_ref.shape[2]
    H2 = w2a_ref.shape[2]
    x = x_ref[0]

    def dot_t(a, b):
        return jax.lax.dot_general(
            a, b, (((1,), (1,)), ((), ())), preferred_element_type=jnp.float32
        )

    b1 = b1_ref[0]
    h_a = dot_t(x, w1a_ref[0, 0]) + b1[:, :I2]
    h_b = dot_t(x, w1b_ref[0, 0]) + b1[:, I2:]
    h_a = 0.5 * h_a * (1.0 + jax.lax.erf(h_a * 0.7071067811865476))
    h_b = 0.5 * h_b * (1.0 + jax.lax.erf(h_b * 0.7071067811865476))

    w2a = w2a_ref[0, 0]
    w2b = w2b_ref[0, 0]
    b2 = b2_ref[0]
    y_a = dot_t(h_a, w2a[:, :I2]) + dot_t(h_b, w2a[:, I2:]) + b2[:, :H2]
    y_b = dot_t(h_a, w2b[:, :I2]) + dot_t(h_b, w2b[:, I2:]) + b2[:, H2:]
    o_ref[0] = jnp.concatenate([y_a, y_b], axis=1)


def kernel(inter_state, W1, b1, W2, b2, loads):
    E, I, H = W1.shape
    CAP = inter_state.shape[0] // E

    x = inter_state.reshape(E, CAP, H)
    W1s = W1.reshape(E, 2, I // 2, H)
    W2s = W2.reshape(E, 2, H // 2, I)
    b1r = b1.reshape(E, 1, I)
    b2r = b2.reshape(E, 1, H)

    out = pl.pallas_call(
        _ffn_kernel,
        grid=(E,),
        in_specs=[
            pl.BlockSpec((1, CAP, H), lambda e: (e, 0, 0)),
            pl.BlockSpec((1, 1, I // 2, H), lambda e: (e, 0, 0, 0)),
            pl.BlockSpec((1, 1, I // 2, H), lambda e: (e, 1, 0, 0)),
            pl.BlockSpec((1, 1, I), lambda e: (e, 0, 0)),
            pl.BlockSpec((1, 1, H // 2, I), lambda e: (e, 0, 0, 0)),
            pl.BlockSpec((1, 1, H // 2, I), lambda e: (e, 1, 0, 0)),
            pl.BlockSpec((1, 1, H), lambda e: (e, 0, 0)),
        ],
        out_specs=pl.BlockSpec((1, CAP, H), lambda e: (e, 0, 0)),
        out_shape=jax.ShapeDtypeStruct((E, CAP, H), jnp.float32),
        compiler_params=pltpu.CompilerParams(
            dimension_semantics=("parallel",),
        ),
    )(x, W1s, W1s, b1r, W2s, W2s, b2r)
    return out.reshape(E * CAP, H)
